# baseline (speedup 1.0000x reference)
import functools

import jax
import jax.numpy as jnp
from jax import lax
from jax.experimental import pallas as pl

N = 10000
E = 320000
D = 128
NUM_GRAPHS = 64
PPAD = 16


def _h0_body(x_ref, w_ref, b_ref, o_ref):
    t = jnp.dot(x_ref[...], w_ref[...], preferred_element_type=jnp.float32)
    t = t + b_ref[...]
    o_ref[...] = t * jax.nn.sigmoid(t)


def _tc_h0(x, w0, b0):
    blk = 1000
    return pl.pallas_call(
        _h0_body,
        grid=(N // blk,),
        in_specs=[
            pl.BlockSpec((blk, D), lambda i: (i, 0)),
            pl.BlockSpec((D, D), lambda i: (0, 0)),
            pl.BlockSpec((1, D), lambda i: (0, 0)),
        ],
        out_specs=pl.BlockSpec((blk, D), lambda i: (i, 0)),
        out_shape=jax.ShapeDtypeStruct((N, D), jnp.float32),
    )(x, w0, b0.reshape(1, D))


def _radial_body(pr_ref, pc_ref, w1_ref, b1_ref, w2_ref, b2_ref, w3_ref,
                 b3_ref, o1_ref, o2_ref, o3_ref):
    d = pr_ref[...] - pc_ref[...]
    d2 = jnp.sum(d * d, axis=1, keepdims=True)
    dist = jnp.sqrt(d2 + 1e-12)
    mu = lax.broadcasted_iota(jnp.int32, (1, PPAD), 1).astype(jnp.float32) * 0.75
    sig = 6.0 / 9.0
    z = (dist - mu) / sig
    rbf = jnp.exp(-(z * z))
    for w_ref, b_ref, o_ref in ((w1_ref, b1_ref, o1_ref),
                                (w2_ref, b2_ref, o2_ref),
                                (w3_ref, b3_ref, o3_ref)):
        t = jnp.dot(rbf, w_ref[...], preferred_element_type=jnp.float32)
        t = t + b_ref[...]
        o_ref[...] = t * jax.nn.sigmoid(t)


def _tc_radial(posr, posc, wc, bc):
    blk = 2000
    wcp = [jnp.pad(w, ((0, PPAD - 9), (0, 0))) for w in wc]
    outs = pl.pallas_call(
        _radial_body,
        grid=(E // blk,),
        in_specs=[
            pl.BlockSpec((blk, PPAD), lambda i: (i, 0)),
            pl.BlockSpec((blk, PPAD), lambda i: (i, 0)),
        ] + [pl.BlockSpec((PPAD, D), lambda i: (0, 0)),
             pl.BlockSpec((1, D), lambda i: (0, 0))] * 3,
        out_specs=[pl.BlockSpec((blk, D), lambda i: (i, 0))] * 3,
        out_shape=[jax.ShapeDtypeStruct((E, D), jnp.float32)] * 3,
    )(posr, posc, wcp[0], bc[0].reshape(1, D), wcp[1], bc[1].reshape(1, D),
      wcp[2], bc[2].reshape(1, D))
    return outs


def _dense_body(h_ref, a_ref, wn_ref, bn_ref, g_ref, be_ref, o_ref):
    t = h_ref[...] + a_ref[0] + a_ref[1]
    t = jnp.dot(t, wn_ref[...], preferred_element_type=jnp.float32)
    t = t + bn_ref[...]
    t = jnp.where(t >= 0, t, 0.01 * t)
    m = jnp.mean(t, axis=0, keepdims=True)
    c = t - m
    v = jnp.mean(c * c, axis=0, keepdims=True)
    o_ref[...] = c / jnp.sqrt(v + 1e-5) * g_ref[...] + be_ref[...]


def _tc_dense(h, agg, wn, bn, g, be):
    return pl.pallas_call(
        _dense_body,
        in_specs=[pl.BlockSpec((N, D), lambda: (0, 0)),
                  pl.BlockSpec((2, N, D), lambda: (0, 0, 0)),
                  pl.BlockSpec((D, D), lambda: (0, 0)),
                  pl.BlockSpec((1, D), lambda: (0, 0)),
                  pl.BlockSpec((1, D), lambda: (0, 0)),
                  pl.BlockSpec((1, D), lambda: (0, 0))],
        out_specs=pl.BlockSpec((N, D), lambda: (0, 0)),
        out_shape=jax.ShapeDtypeStruct((N, D), jnp.float32),
    )(h, agg, wn, bn.reshape(1, D), g.reshape(1, D), be.reshape(1, D))


def _graphsum_body(h_ref, b_ref, o_ref):
    onehot = (b_ref[...] == lax.broadcasted_iota(jnp.int32, (N, NUM_GRAPHS),
                                                 1)).astype(jnp.float32)
    o_ref[...] = lax.dot_general(onehot, h_ref[...], (((0,), (0,)), ((), ())),
                                 preferred_element_type=jnp.float32)


def _tc_graphsum(h, batch):
    return pl.pallas_call(
        _graphsum_body,
        in_specs=[pl.BlockSpec((N, D), lambda: (0, 0)),
                  pl.BlockSpec((N, 1), lambda: (0, 0))],
        out_specs=pl.BlockSpec((NUM_GRAPHS, D), lambda: (0, 0)),
        out_shape=jax.ShapeDtypeStruct((NUM_GRAPHS, D), jnp.float32),
    )(h, batch.reshape(N, 1))


def _gather_pos(pos16, row, col):
    return pos16[row], pos16[col]


def _edge_pass(h, radial, row, col):
    msg = h[row] * radial
    agg = jax.ops.segment_sum(msg, col, num_segments=N)
    return jnp.stack([agg, jnp.zeros_like(agg)])


def kernel(x, edge_index, pos, edge_attr, batch, W0, b0,
           Wc1, bc1, Wn1, bn1, g1, be1,
           Wc2, bc2, Wn2, bn2, g2, be2,
           Wc3, bc3, Wn3, bn3, g3, be3):
    row = edge_index[0].astype(jnp.int32)
    col = edge_index[1].astype(jnp.int32)
    pos16 = jnp.pad(pos, ((0, 0), (0, PPAD - 3)))
    posr, posc = _gather_pos(pos16, row, col)
    h = _tc_h0(x, W0, b0)
    r1, r2, r3 = _tc_radial(posr, posc, (Wc1, Wc2, Wc3), (bc1, bc2, bc3))
    for radial, wn, bn, g, be in ((r1, Wn1, bn1, g1, be1),
                                  (r2, Wn2, bn2, g2, be2),
                                  (r3, Wn3, bn3, g3, be3)):
        agg = _edge_pass(h, radial, row, col)
        h = _tc_dense(h, agg, wn, bn, g, be)
    return _tc_graphsum(h, batch)


# speedup vs baseline: 3.1355x; 3.1355x over previous
import functools

import jax
import jax.numpy as jnp
from jax import lax
from jax.experimental import pallas as pl
from jax.experimental.pallas import tpu as pltpu
from jax.experimental.pallas import tpu_sc as plsc

N = 10000
E = 320000
D = 128
NUM_GRAPHS = 64
PPAD = 16

_NC, _NS = 2, 16
_NW = _NC * _NS
_W = 128
_NWIN = E // _W
_KMAX = (_NWIN + _NW - 1) // _NW
_RPS = 624
_CHUNKS = ((0, 128), (128, 128), (256, 128), (384, 128), (512, 112))
_ZR = 128


def _h0_body(x_ref, w_ref, b_ref, o_ref):
    t = jnp.dot(x_ref[...], w_ref[...], preferred_element_type=jnp.float32)
    t = t + b_ref[...]
    o_ref[...] = t * jax.nn.sigmoid(t)


def _tc_h0(x, w0, b0):
    blk = 1000
    return pl.pallas_call(
        _h0_body,
        grid=(N // blk,),
        in_specs=[
            pl.BlockSpec((blk, D), lambda i: (i, 0)),
            pl.BlockSpec((D, D), lambda i: (0, 0)),
            pl.BlockSpec((1, D), lambda i: (0, 0)),
        ],
        out_specs=pl.BlockSpec((blk, D), lambda i: (i, 0)),
        out_shape=jax.ShapeDtypeStruct((N, D), jnp.float32),
    )(x, w0, b0.reshape(1, D))


def _radial_body(dif_ref, w1_ref, b1_ref, w2_ref, b2_ref, w3_ref,
                 b3_ref, o1_ref, o2_ref, o3_ref):
    d = dif_ref[...]
    d2 = jnp.sum(d * d, axis=1, keepdims=True)
    dist = jnp.sqrt(d2 + 1e-12)
    mu = lax.broadcasted_iota(jnp.int32, (1, PPAD), 1).astype(jnp.float32) * 0.75
    sig = 6.0 / 9.0
    z = (dist - mu) / sig
    rbf = jnp.exp(-(z * z))
    for w_ref, b_ref, o_ref in ((w1_ref, b1_ref, o1_ref),
                                (w2_ref, b2_ref, o2_ref),
                                (w3_ref, b3_ref, o3_ref)):
        t = jnp.dot(rbf, w_ref[...], preferred_element_type=jnp.float32)
        t = t + b_ref[...]
        o_ref[...] = t * jax.nn.sigmoid(t)


def _tc_radial(dif, wc, bc):
    blk = 2000
    wcp = [jnp.pad(w, ((0, PPAD - 9), (0, 0))) for w in wc]
    outs = pl.pallas_call(
        _radial_body,
        grid=(E // blk,),
        in_specs=[
            pl.BlockSpec((blk, D), lambda i: (i, 0)),
        ] + [pl.BlockSpec((PPAD, D), lambda i: (0, 0)),
             pl.BlockSpec((1, D), lambda i: (0, 0))] * 3,
        out_specs=[pl.BlockSpec((blk, D), lambda i: (i, 0))] * 3,
        out_shape=[jax.ShapeDtypeStruct((E, D), jnp.float32)] * 3,
    )(dif, wcp[0], bc[0].reshape(1, D), wcp[1], bc[1].reshape(1, D),
      wcp[2], bc[2].reshape(1, D))
    return outs


def _dense_body(h_ref, a_ref, wn_ref, bn_ref, g_ref, be_ref, o_ref):
    t = h_ref[...] + a_ref[0] + a_ref[1]
    t = jnp.dot(t, wn_ref[...], preferred_element_type=jnp.float32)
    t = t + bn_ref[...]
    t = jnp.where(t >= 0, t, 0.01 * t)
    m = jnp.mean(t, axis=0, keepdims=True)
    c = t - m
    v = jnp.mean(c * c, axis=0, keepdims=True)
    o_ref[...] = c / jnp.sqrt(v + 1e-5) * g_ref[...] + be_ref[...]


def _tc_dense(h, agg, wn, bn, g, be):
    return pl.pallas_call(
        _dense_body,
        in_specs=[pl.BlockSpec((N, D), lambda: (0, 0)),
                  pl.BlockSpec((2, N, D), lambda: (0, 0, 0)),
                  pl.BlockSpec((D, D), lambda: (0, 0)),
                  pl.BlockSpec((1, D), lambda: (0, 0)),
                  pl.BlockSpec((1, D), lambda: (0, 0)),
                  pl.BlockSpec((1, D), lambda: (0, 0))],
        out_specs=pl.BlockSpec((N, D), lambda: (0, 0)),
        out_shape=jax.ShapeDtypeStruct((N, D), jnp.float32),
    )(h, agg, wn, bn.reshape(1, D), g.reshape(1, D), be.reshape(1, D))


def _graphsum_body(h_ref, b_ref, o_ref):
    onehot = (b_ref[...] == lax.broadcasted_iota(jnp.int32, (N, NUM_GRAPHS),
                                                 1)).astype(jnp.float32)
    o_ref[...] = lax.dot_general(onehot, h_ref[...], (((0,), (0,)), ((), ())),
                                 preferred_element_type=jnp.float32)


def _tc_graphsum(h, batch):
    return pl.pallas_call(
        _graphsum_body,
        in_specs=[pl.BlockSpec((N, D), lambda: (0, 0)),
                  pl.BlockSpec((N, 1), lambda: (0, 0))],
        out_specs=pl.BlockSpec((NUM_GRAPHS, D), lambda: (0, 0)),
        out_shape=jax.ShapeDtypeStruct((NUM_GRAPHS, D), jnp.float32),
    )(h, batch.reshape(N, 1))


def _posg_body(pos_hbm, row_hbm, col_hbm, od_hbm,
               ridx, cidx, bufr, bufc, sem):
    c = lax.axis_index("c")
    s = lax.axis_index("s")
    wid = s * _NC + c

    @pl.loop(0, _KMAX)
    def _win(k):
        wi = wid + _NW * k

        @pl.when(wi < _NWIN)
        def _():
            pltpu.sync_copy(row_hbm.at[pl.ds(wi * _W, _W)], ridx)
            pltpu.sync_copy(col_hbm.at[pl.ds(wi * _W, _W)], cidx)
            pltpu.async_copy(pos_hbm.at[ridx], bufr, sem).wait()
            pltpu.async_copy(pos_hbm.at[cidx], bufc, sem).wait()

            @pl.loop(0, _W)
            def _diff(r):
                for j in range(8):
                    sl = (r, pl.ds(j * 16, 16))
                    bufr[sl] = bufr[sl] - bufc[sl]

            pltpu.sync_copy(bufr, od_hbm.at[pl.ds(wi * _W, _W), :])


def _sc_pos_diff(pos128, row2d, col2d):
    mesh = plsc.VectorSubcoreMesh(core_axis_name="c", subcore_axis_name="s")
    f = pl.kernel(
        _posg_body,
        out_type=jax.ShapeDtypeStruct((E, D), jnp.float32),
        mesh=mesh,
        scratch_types=[
            pltpu.VMEM((_W,), jnp.int32),
            pltpu.VMEM((_W,), jnp.int32),
            pltpu.VMEM((_W, D), jnp.float32),
            pltpu.VMEM((_W, D), jnp.float32),
            pltpu.SemaphoreType.DMA,
        ],
    )
    return f(pos128, row2d, col2d)


def _edge_body(h_hbm, rad_hbm, row_hbm, col_hbm, out_hbm,
               aggs, ridx, cidx, gath, rad, zbuf, sem):
    c = lax.axis_index("c")
    s = lax.axis_index("s")
    wid = s * _NC + c

    z16 = jnp.zeros((16,), jnp.float32)

    @pl.loop(0, _ZR)
    def _zb(r):
        for j in range(8):
            zbuf[r, pl.ds(j * 16, 16)] = z16

    base = s * _RPS
    for off, sz in _CHUNKS:
        pltpu.sync_copy(zbuf.at[pl.ds(0, sz), :],
                        aggs.at[pl.ds(base + off, sz), :])

    @pl.when(s == _NS - 1)
    def _ztail():
        pltpu.sync_copy(zbuf.at[pl.ds(0, 16), :],
                        aggs.at[pl.ds(_NS * _RPS, 16), :])

    plsc.subcore_barrier()

    @pl.loop(0, _KMAX)
    def _win(k):
        wi = wid + _NW * k

        @pl.when(wi < _NWIN)
        def _():
            pltpu.sync_copy(row_hbm.at[pl.ds(wi * _W, _W)], ridx)
            pltpu.sync_copy(col_hbm.at[pl.ds(wi * _W, _W)], cidx)
            pltpu.async_copy(h_hbm.at[ridx], gath, sem).wait()
            pltpu.sync_copy(rad_hbm.at[pl.ds(wi * _W, _W), :], rad)

            @pl.loop(0, _W)
            def _mul(r):
                for j in range(8):
                    sl = (r, pl.ds(j * 16, 16))
                    gath[sl] = gath[sl] * rad[sl]

            pltpu.sync_copy(gath, aggs.at[cidx], add=True)

    plsc.subcore_barrier()
    for off, sz in _CHUNKS:
        pltpu.sync_copy(aggs.at[pl.ds(base + off, sz), :],
                        out_hbm.at[c, pl.ds(base + off, sz), :])

    @pl.when(s == _NS - 1)
    def _wtail():
        pltpu.sync_copy(aggs.at[pl.ds(_NS * _RPS, 16), :],
                        out_hbm.at[c, pl.ds(_NS * _RPS, 16), :])


def _sc_edge_pass(h, radial, row2d, col2d):
    mesh = plsc.VectorSubcoreMesh(core_axis_name="c", subcore_axis_name="s")
    f = pl.kernel(
        _edge_body,
        out_type=jax.ShapeDtypeStruct((_NC, N, D), jnp.float32),
        mesh=mesh,
        scratch_types=[
            pltpu.VMEM_SHARED((N, D), jnp.float32),
            pltpu.VMEM((_W,), jnp.int32),
            pltpu.VMEM((_W,), jnp.int32),
            pltpu.VMEM((_W, D), jnp.float32),
            pltpu.VMEM((_W, D), jnp.float32),
            pltpu.VMEM((_ZR, D), jnp.float32),
            pltpu.SemaphoreType.DMA,
        ],
    )
    return f(h, radial, row2d, col2d)


def kernel(x, edge_index, pos, edge_attr, batch, W0, b0,
           Wc1, bc1, Wn1, bn1, g1, be1,
           Wc2, bc2, Wn2, bn2, g2, be2,
           Wc3, bc3, Wn3, bn3, g3, be3):
    row = edge_index[0].astype(jnp.int32)
    col = edge_index[1].astype(jnp.int32)
    pos128 = jnp.pad(pos, ((0, 0), (0, D - 3)))
    dif = _sc_pos_diff(pos128, row, col)
    h = _tc_h0(x, W0, b0)
    r1, r2, r3 = _tc_radial(dif, (Wc1, Wc2, Wc3), (bc1, bc2, bc3))
    for radial, wn, bn, g, be in ((r1, Wn1, bn1, g1, be1),
                                  (r2, Wn2, bn2, g2, be2),
                                  (r3, Wn3, bn3, g3, be3)):
        agg = _sc_edge_pass(h, radial, row, col)
        h = _tc_dense(h, agg, wn, bn, g, be)
    return _tc_graphsum(h, batch)
